# baseline (speedup 1.0000x reference)
import functools

import jax
import jax.numpy as jnp
from jax import lax
from jax.experimental import pallas as pl
from jax.experimental.pallas import tpu as pltpu
from jax.experimental.pallas import tpu_sc as plsc

B = 32
N = 512 * 512
NC, NS, L = 2, 16, 16
CHUNK = 16384
NCHUNK = N // CHUNK


def _log1p_exp_neg(e):
    z = e / (2.0 + e)
    u = z * z
    p = 2.0 / 9.0 + u * (2.0 / 11.0)
    p = 2.0 / 7.0 + u * p
    p = 2.0 / 5.0 + u * p
    p = 2.0 / 3.0 + u * p
    p = 2.0 + u * p
    return z * p


def _sc_body(x_hbm, t_hbm, out_hbm, xbuf, tbuf, obuf):
    c = lax.axis_index("c")
    s = lax.axis_index("s")
    w = s * NC + c

    zero = jnp.zeros((L,), jnp.float32)
    acc = (zero, zero, zero, zero)

    def inner(i, carry):
        sacc, stacc, tacc, bacc = carry
        xv = xbuf[pl.ds(i * L, L)]
        tv = tbuf[pl.ds(i * L, L)].astype(jnp.float32)
        ax = jnp.abs(xv)
        e = jnp.exp(-ax)
        inv = 1.0 / (1.0 + e)
        sig = jnp.where(xv >= 0.0, inv, e * inv)
        bce = jnp.maximum(xv, 0.0) - xv * tv + _log1p_exp_neg(e)
        return (sacc + sig, stacc + sig * tv, tacc + tv, bacc + bce)

    for chunk in range(NCHUNK):
        pltpu.sync_copy(x_hbm.at[w, chunk], xbuf)
        pltpu.sync_copy(t_hbm.at[w, chunk], tbuf)
        acc = lax.fori_loop(0, CHUNK // L, inner, acc)

    for k in range(4):
        obuf[pl.ds(k * L, L)] = acc[k]
    pltpu.sync_copy(obuf, out_hbm.at[w])


@functools.partial(
    pl.kernel,
    out_type=jax.ShapeDtypeStruct((B, 4 * L), jnp.float32),
    mesh=plsc.VectorSubcoreMesh(
        core_axis_name="c", subcore_axis_name="s",
        num_cores=NC, num_subcores=NS),
    scratch_types=[
        pltpu.VMEM((CHUNK,), jnp.float32),
        pltpu.VMEM((CHUNK,), jnp.int32),
        pltpu.VMEM((4 * L,), jnp.float32),
    ],
)
def _sc_partials(x_hbm, t_hbm, out_hbm, xbuf, tbuf, obuf):
    _sc_body(x_hbm, t_hbm, out_hbm, xbuf, tbuf, obuf)


@jax.jit
def kernel(inputs, target):
    x = inputs.reshape(B, NCHUNK, CHUNK)
    t = target.reshape(B, NCHUNK, CHUNK)
    parts = _sc_partials(x, t).reshape(B, 4, L).sum(axis=2)
    s_sum = parts[:, 0]
    st_sum = parts[:, 1]
    t_sum = parts[:, 2]
    b_sum = parts[:, 3]
    dice = 1.0 - (2.0 * st_sum + 1.0) / (s_sum + t_sum + 1.0)
    loss = jnp.mean(dice) + jnp.sum(b_sum) / (B * N)
    return loss.reshape(1)


# speedup vs baseline: 1.4578x; 1.4578x over previous
import functools

import jax
import jax.numpy as jnp
from jax import lax
from jax.experimental import pallas as pl
from jax.experimental.pallas import tpu as pltpu
from jax.experimental.pallas import tpu_sc as plsc

B = 32
N = 512 * 512
NC, NS, L = 2, 16, 16
NW = NC * NS
CHUNK = 16384
NCHUNK = N // CHUNK

SC_B = 8
K = NW // SC_B
WCHUNKS = NCHUNK // K
UNROLL = 4

SPB = 4
TC_B = B - SC_B
TC_GRID = TC_B // SPB


def _log1p_poly(z):
    u = z * z
    p = 2.0 / 9.0 + u * (2.0 / 11.0)
    p = 2.0 / 7.0 + u * p
    p = 2.0 / 5.0 + u * p
    p = 2.0 / 3.0 + u * p
    p = 2.0 + u * p
    return z * p


def _sc_body(x_hbm, t_hbm, out_hbm, xbuf, tbuf, obuf):
    c = lax.axis_index("c")
    s = lax.axis_index("s")
    w = s * NC + c
    sid = w // K
    chunk0 = (w % K) * WCHUNKS

    zero = jnp.zeros((L,), jnp.float32)
    acc = (zero, zero, zero, zero)

    def inner(i, carry):
        sacc, stacc, tacc, bacc = carry
        for u in range(UNROLL):
            off = (i * UNROLL + u) * L
            xv = xbuf[pl.ds(off, L)]
            tv = tbuf[pl.ds(off, L)].astype(jnp.float32)
            ax = jnp.abs(xv)
            e = jnp.exp(-ax)
            a = 1.0 + e
            b = 2.0 + e
            q = 1.0 / (a * b)
            inv = q * b
            z = (e * q) * a
            sig = jnp.where(xv >= 0.0, inv, e * inv)
            bce = jnp.maximum(xv, 0.0) - xv * tv + _log1p_poly(z)
            sacc = sacc + sig
            stacc = stacc + sig * tv
            tacc = tacc + tv
            bacc = bacc + bce
        return (sacc, stacc, tacc, bacc)

    for chunk in range(WCHUNKS):
        pltpu.sync_copy(x_hbm.at[sid, chunk0 + chunk], xbuf)
        pltpu.sync_copy(t_hbm.at[sid, chunk0 + chunk], tbuf)
        acc = lax.fori_loop(0, CHUNK // (L * UNROLL), inner, acc)

    for k in range(4):
        obuf[pl.ds(k * L, L)] = acc[k]
    pltpu.sync_copy(obuf, out_hbm.at[w])


_sc_partials = functools.partial(
    pl.kernel,
    out_type=jax.ShapeDtypeStruct((NW, 4 * L), jnp.float32),
    mesh=plsc.VectorSubcoreMesh(
        core_axis_name="c", subcore_axis_name="s",
        num_cores=NC, num_subcores=NS),
    scratch_types=[
        pltpu.VMEM((CHUNK,), jnp.float32),
        pltpu.VMEM((CHUNK,), jnp.int32),
        pltpu.VMEM((4 * L,), jnp.float32),
    ],
)(_sc_body)


def _tc_body(x_ref, t_ref, out_ref):
    x = x_ref[...].reshape(SPB, N)
    t = t_ref[...].reshape(SPB, N).astype(jnp.float32)

    ax = jnp.abs(x)
    e = jnp.exp(-ax)
    inv = 1.0 / (1.0 + e)
    sig = jnp.where(x >= 0.0, inv, e * inv)
    bce = jnp.maximum(x, 0.0) - x * t + jnp.log1p(e)

    s_sum = jnp.sum(sig, axis=1)
    st_sum = jnp.sum(sig * t, axis=1)
    t_sum = jnp.sum(t, axis=1)
    b_sum = jnp.sum(bce, axis=1)

    col = lax.broadcasted_iota(jnp.int32, (1, SPB, 128), 2)
    out_ref[...] = (jnp.where(col == 0, s_sum[None, :, None], 0.0)
                    + jnp.where(col == 1, st_sum[None, :, None], 0.0)
                    + jnp.where(col == 2, t_sum[None, :, None], 0.0)
                    + jnp.where(col == 3, b_sum[None, :, None], 0.0))


def _tc_partials(inputs, target):
    return pl.pallas_call(
        _tc_body,
        grid=(TC_GRID,),
        in_specs=[
            pl.BlockSpec((SPB, 1, 512, 512), lambda i: (SC_B // SPB + i, 0, 0, 0)),
            pl.BlockSpec((SPB, 512, 512), lambda i: (SC_B // SPB + i, 0, 0)),
        ],
        out_specs=pl.BlockSpec((1, SPB, 128), lambda i: (i, 0, 0)),
        out_shape=jax.ShapeDtypeStruct((TC_GRID, SPB, 128), jnp.float32),
        compiler_params=pltpu.CompilerParams(
            dimension_semantics=("arbitrary",),
        ),
    )(inputs, target)


@jax.jit
def kernel(inputs, target):
    x3 = inputs.reshape(B, NCHUNK, CHUNK)
    t3 = target.reshape(B, NCHUNK, CHUNK)
    sc_w = _sc_partials(x3, t3).reshape(NW, 4, L).sum(axis=2)
    sc_parts = sc_w.reshape(SC_B, K, 4).sum(axis=1)
    tc_parts = _tc_partials(inputs, target).reshape(TC_B, 128)[:, :4]
    parts = jnp.concatenate([sc_parts, tc_parts], axis=0)
    s_sum = parts[:, 0]
    st_sum = parts[:, 1]
    t_sum = parts[:, 2]
    b_sum = parts[:, 3]
    dice = 1.0 - (2.0 * st_sum + 1.0) / (s_sum + t_sum + 1.0)
    loss = jnp.mean(dice) + jnp.sum(b_sum) / (B * N)
    return loss.reshape(1)


# speedup vs baseline: 2.5901x; 1.7767x over previous
import functools

import jax
import jax.numpy as jnp
from jax import lax
from jax.experimental import pallas as pl
from jax.experimental.pallas import tpu as pltpu
from jax.experimental.pallas import tpu_sc as plsc

B = 32
N = 512 * 512
NC, NS, L = 2, 16, 16
NW = NC * NS

SC_B = 8
K = NW // SC_B
WROWS = 512 // K
SLAB = 32
NSLAB = WROWS // SLAB
ROWV = 512 // L

SPB = 4
TC_B = B - SC_B
TC_GRID = TC_B // SPB


def _log1p_poly(z):
    u = z * z
    p = 2.0 / 9.0 + u * (2.0 / 11.0)
    p = 2.0 / 7.0 + u * p
    p = 2.0 / 5.0 + u * p
    p = 2.0 / 3.0 + u * p
    p = 2.0 + u * p
    return z * p


def _sc_body(x_hbm, t_hbm, out_hbm, xbuf, tbuf, obuf):
    c = lax.axis_index("c")
    s = lax.axis_index("s")
    w = s * NC + c
    sid = w // K
    row0 = (w % K) * WROWS

    zero = jnp.zeros((L,), jnp.float32)
    acc = (zero, zero, zero, zero)

    def inner(r, carry):
        sacc, stacc, tacc, bacc = carry
        for u in range(ROWV):
            xv = xbuf[r, pl.ds(u * L, L)]
            tv = tbuf[r, pl.ds(u * L, L)].astype(jnp.float32)
            ax = jnp.abs(xv)
            e = jnp.exp(-ax)
            a = 1.0 + e
            b = 2.0 + e
            q = 1.0 / (a * b)
            inv = q * b
            z = (e * q) * a
            sig = jnp.where(xv >= 0.0, inv, e * inv)
            bce = jnp.maximum(xv, 0.0) - xv * tv + _log1p_poly(z)
            sacc = sacc + sig
            stacc = stacc + sig * tv
            tacc = tacc + tv
            bacc = bacc + bce
        return (sacc, stacc, tacc, bacc)

    for slab in range(NSLAB):
        pltpu.sync_copy(x_hbm.at[sid, pl.ds(row0 + slab * SLAB, SLAB)], xbuf)
        pltpu.sync_copy(t_hbm.at[sid, pl.ds(row0 + slab * SLAB, SLAB)], tbuf)
        acc = lax.fori_loop(0, SLAB, inner, acc)

    for k in range(4):
        obuf[pl.ds(k * L, L)] = acc[k]
    pltpu.sync_copy(obuf, out_hbm.at[w])


_sc_partials = functools.partial(
    pl.kernel,
    out_type=jax.ShapeDtypeStruct((NW, 128), jnp.float32),
    mesh=plsc.VectorSubcoreMesh(
        core_axis_name="c", subcore_axis_name="s",
        num_cores=NC, num_subcores=NS),
    scratch_types=[
        pltpu.VMEM((SLAB, 512), jnp.float32),
        pltpu.VMEM((SLAB, 512), jnp.int32),
        pltpu.VMEM((128,), jnp.float32),
    ],
    compiler_params=pltpu.CompilerParams(use_tc_tiling_on_sc=True),
)(_sc_body)


def _tc_body(x_ref, t_ref, out_ref):
    x = x_ref[...].reshape(SPB, N)
    t = t_ref[...].reshape(SPB, N).astype(jnp.float32)

    ax = jnp.abs(x)
    e = jnp.exp(-ax)
    inv = 1.0 / (1.0 + e)
    sig = jnp.where(x >= 0.0, inv, e * inv)
    bce = jnp.maximum(x, 0.0) - x * t + jnp.log1p(e)

    s_sum = jnp.sum(sig, axis=1)
    st_sum = jnp.sum(sig * t, axis=1)
    t_sum = jnp.sum(t, axis=1)
    b_sum = jnp.sum(bce, axis=1)

    col = lax.broadcasted_iota(jnp.int32, (1, SPB, 128), 2)
    out_ref[...] = (jnp.where(col == 0, s_sum[None, :, None], 0.0)
                    + jnp.where(col == 1, st_sum[None, :, None], 0.0)
                    + jnp.where(col == 2, t_sum[None, :, None], 0.0)
                    + jnp.where(col == 3, b_sum[None, :, None], 0.0))


def _tc_partials(inputs, target):
    return pl.pallas_call(
        _tc_body,
        grid=(TC_GRID,),
        in_specs=[
            pl.BlockSpec((SPB, 1, 512, 512), lambda i: (SC_B // SPB + i, 0, 0, 0)),
            pl.BlockSpec((SPB, 512, 512), lambda i: (SC_B // SPB + i, 0, 0)),
        ],
        out_specs=pl.BlockSpec((1, SPB, 128), lambda i: (i, 0, 0)),
        out_shape=jax.ShapeDtypeStruct((TC_GRID, SPB, 128), jnp.float32),
        compiler_params=pltpu.CompilerParams(
            dimension_semantics=("arbitrary",),
        ),
    )(inputs, target)


@jax.jit
def kernel(inputs, target):
    x3 = inputs.reshape(B, 512, 512)
    sc_w = _sc_partials(x3, target)[:, :64].reshape(NW, 4, L).sum(axis=2)
    sc_parts = sc_w.reshape(SC_B, K, 4).sum(axis=1)
    tc_parts = _tc_partials(inputs, target).reshape(TC_B, 128)[:, :4]
    parts = jnp.concatenate([sc_parts, tc_parts], axis=0)
    s_sum = parts[:, 0]
    st_sum = parts[:, 1]
    t_sum = parts[:, 2]
    b_sum = parts[:, 3]
    dice = 1.0 - (2.0 * st_sum + 1.0) / (s_sum + t_sum + 1.0)
    loss = jnp.mean(dice) + jnp.sum(b_sum) / (B * N)
    return loss.reshape(1)
